# baseline (speedup 1.0000x reference)
import functools

import jax
import jax.numpy as jnp
from jax import lax
from jax.experimental import pallas as pl
from jax.experimental.pallas import tpu as pltpu
from jax.experimental.pallas import tpu_sc as plsc

_N_ATOMS = 10000
_N_PAD = 10240
_CHUNK = 80
_SPLITS = (29, 29, 29, 29, 9)


def _swish(v):
    return v * jax.nn.sigmoid(v)


def _edge_body(x_ref, rbft_ref, wx_ref, bx_ref, wr_ref, br_ref, out_ref):
    h = jnp.dot(x_ref[...], wx_ref[...], preferred_element_type=jnp.float32)
    h = _swish(h + bx_ref[...])
    r = lax.dot_general(rbft_ref[...], wr_ref[...],
                        dimension_numbers=(((0,), (0,)), ((), ())),
                        preferred_element_type=jnp.float32)
    r = _swish(r + br_ref[...])
    out_ref[...] = h * r


def _edge_stage(x, rbf_t, W_x, b_x, W_rbf, b_rbf, off_blocks, n_blocks,
                block_e):
    E, EMB = x.shape
    RBF = rbf_t.shape[0]
    return pl.pallas_call(
        _edge_body,
        grid=(n_blocks,),
        in_specs=[
            pl.BlockSpec((block_e, EMB), lambda i: (off_blocks + i, 0)),
            pl.BlockSpec((RBF, block_e), lambda i: (0, off_blocks + i)),
            pl.BlockSpec((EMB, EMB), lambda i: (0, 0)),
            pl.BlockSpec((1, EMB), lambda i: (0, 0)),
            pl.BlockSpec((RBF, EMB), lambda i: (0, 0)),
            pl.BlockSpec((1, EMB), lambda i: (0, 0)),
        ],
        out_specs=pl.BlockSpec((block_e, EMB), lambda i: (i, 0)),
        out_shape=jax.ShapeDtypeStruct((n_blocks * block_e, EMB), jnp.float32),
    )(x, rbf_t, W_x, b_x.reshape(1, EMB), W_rbf, b_rbf.reshape(1, EMB))


def _seg_sum_sc(conv3, idnb, chunk_off):
    n_chunks, _, EMB = conv3.shape
    info = plsc.get_sparse_core_info()
    NC, NS = info.num_cores, info.num_subcores
    nw = NC * NS
    cpw = n_chunks // nw
    rows_per_s = _N_PAD // NS
    mesh = plsc.VectorSubcoreMesh(core_axis_name="c", subcore_axis_name="s")
    nbuf = 4
    n_groups = cpw // nbuf
    n_tail = cpw - n_groups * nbuf

    @functools.partial(
        pl.kernel,
        mesh=mesh,
        out_type=jax.ShapeDtypeStruct((NC, _N_PAD, EMB), jnp.float32),
        scratch_types=[
            pltpu.VMEM((nbuf, _CHUNK), jnp.int32),
            pltpu.VMEM((nbuf, _CHUNK, EMB), jnp.float32),
            pltpu.VMEM_SHARED((_N_PAD, EMB), jnp.float32),
            pltpu.SemaphoreType.DMA((nbuf,)),
            pltpu.SemaphoreType.DMA((nbuf,)),
            pltpu.SemaphoreType.DMA((nbuf,)),
        ],
    )
    def k(conv_hbm, idnb_hbm, out_hbm, idx_ring, bufs, acc, semf, semi,
          sems):
        c = lax.axis_index("c")
        s = lax.axis_index("s")
        w = c * NS + s
        base = w * cpw
        ebase = (chunk_off + w * cpw) * _CHUNK
        my_rows = pl.ds(s * rows_per_s, rows_per_s)
        zero16 = jnp.zeros((16,), jnp.float32)

        def zfill(r, carry):
            for c16 in range(EMB // 16):
                bufs[0, r, pl.ds(c16 * 16, 16)] = zero16
            return carry

        lax.fori_loop(0, _CHUNK, zfill, None)
        for r in range(rows_per_s // _CHUNK):
            pltpu.sync_copy(
                bufs.at[0], acc.at[pl.ds(s * rows_per_s + r * _CHUNK, _CHUNK)])
        plsc.subcore_barrier()

        for b in range(nbuf):
            pltpu.async_copy(conv_hbm.at[base + b], bufs.at[b], semf.at[b])
            pltpu.async_copy(idnb_hbm.at[pl.ds(ebase + b * _CHUNK, _CHUNK)],
                             idx_ring.at[b], semi.at[b])

        def body(g, carry):
            j0 = nbuf * g
            for b in range(nbuf):
                pltpu.make_async_copy(
                    conv_hbm.at[base], bufs.at[b], semf.at[b]).wait()
                pltpu.make_async_copy(
                    idnb_hbm.at[pl.ds(0, _CHUNK)], idx_ring.at[b],
                    semi.at[b]).wait()
                pltpu.async_copy(
                    bufs.at[b], acc.at[idx_ring.at[b]], sems.at[b],
                    add=True)
            for b in range(nbuf):
                pltpu.make_async_copy(
                    bufs.at[b], acc.at[idx_ring.at[b]], sems.at[b]).wait()
                nxt = j0 + nbuf + b

                @pl.when(nxt < cpw)
                def _():
                    pltpu.async_copy(
                        conv_hbm.at[base + nxt], bufs.at[b], semf.at[b])
                    pltpu.async_copy(
                        idnb_hbm.at[pl.ds(ebase + nxt * _CHUNK, _CHUNK)],
                        idx_ring.at[b], semi.at[b])
            return carry

        lax.fori_loop(0, n_groups, body, None)
        for t in range(n_tail):
            pltpu.make_async_copy(
                conv_hbm.at[base], bufs.at[t], semf.at[t]).wait()
            pltpu.make_async_copy(
                idnb_hbm.at[pl.ds(0, _CHUNK)], idx_ring.at[t],
                semi.at[t]).wait()
            pltpu.sync_copy(bufs.at[t], acc.at[idx_ring.at[t]], add=True)
        plsc.subcore_barrier()
        pltpu.sync_copy(acc.at[my_rows], out_hbm.at[c].at[my_rows])

    return k(conv3, idnb)


def _node_body(*refs):
    nk = len(_SPLITS)
    p_refs = refs[:nk]
    w1_ref, b1_ref, w2_ref, b2_ref, w3_ref, b3_ref, wf_ref, out_ref = refs[nk:]
    h = p_refs[0][0] + p_refs[0][1]
    for p in p_refs[1:]:
        h = h + p[0] + p[1]
    h = _swish(jnp.dot(h, w1_ref[...], preferred_element_type=jnp.float32)
               + b1_ref[...])
    h = _swish(jnp.dot(h, w2_ref[...], preferred_element_type=jnp.float32)
               + b2_ref[...])
    h = _swish(jnp.dot(h, w3_ref[...], preferred_element_type=jnp.float32)
               + b3_ref[...])
    out_ref[...] = lax.dot_general(wf_ref[...], h,
                                   dimension_numbers=(((0,), (1,)), ((), ())),
                                   preferred_element_type=jnp.float32)


def _node_stage(partials, W1, b1, W2, b2, W3, b3, W_final, n_out, block_n):
    NC, _, EMB = partials[0].shape
    nk = len(_SPLITS)
    grid = (n_out // block_n,)
    return pl.pallas_call(
        _node_body,
        grid=grid,
        in_specs=[
            pl.BlockSpec((NC, block_n, EMB), lambda i: (0, i, 0))
            for _ in range(nk)
        ] + [
            pl.BlockSpec((EMB, EMB), lambda i: (0, 0)),
            pl.BlockSpec((1, EMB), lambda i: (0, 0)),
            pl.BlockSpec((EMB, EMB), lambda i: (0, 0)),
            pl.BlockSpec((1, EMB), lambda i: (0, 0)),
            pl.BlockSpec((EMB, EMB), lambda i: (0, 0)),
            pl.BlockSpec((1, EMB), lambda i: (0, 0)),
            pl.BlockSpec((EMB, 1), lambda i: (0, 0)),
        ],
        out_specs=pl.BlockSpec((1, block_n), lambda i: (0, i)),
        out_shape=jax.ShapeDtypeStruct((1, n_out), jnp.float32),
    )(*partials, W1, b1.reshape(1, EMB), W2, b2.reshape(1, EMB),
      W3, b3.reshape(1, EMB), W_final)


def kernel(x, rbf, idnb_i, n_atoms, coef_rbf_a, coef_rbf_b, coef_x,
           coef_final, W_x, b_x, W_rbf, b_rbf, W1, b1, W2, b2, W3, b3,
           W_final):
    E, EMB = x.shape
    W_rbf_s = W_rbf * coef_rbf_a[0]
    W1_s = W1 * (coef_x[0] * coef_rbf_b[0])
    W_final_s = W_final * coef_final[0]

    nw = 32
    block_e = nw * _CHUNK
    rbf_t = rbf.T

    partials = []
    cum = 0
    for cpw in _SPLITS:
        conv = _edge_stage(x, rbf_t, W_x, b_x, W_rbf_s, b_rbf,
                           off_blocks=cum, n_blocks=cpw, block_e=block_e)
        conv3 = conv.reshape(cpw * nw, _CHUNK, EMB)
        partials.append(_seg_sum_sc(conv3, idnb_i, chunk_off=cum * nw))
        cum += cpw

    out_t = _node_stage(partials, W1_s, b1, W2, b2, W3, b3, W_final_s,
                        n_out=_N_PAD, block_n=2048)
    return out_t[:, :_N_ATOMS].T


# speedup vs baseline: 1.0749x; 1.0749x over previous
import functools

import jax
import jax.numpy as jnp
from jax import lax
from jax.experimental import pallas as pl
from jax.experimental.pallas import tpu as pltpu
from jax.experimental.pallas import tpu_sc as plsc

_N_ATOMS = 10000
_N_PAD = 10240
_CHUNK = 80
_SPLITS = (30, 28, 24, 22, 21)


def _swish(v):
    return v * jax.nn.sigmoid(v)


def _edge_body(x_ref, rbft_ref, wx_ref, bx_ref, wr_ref, br_ref, out_ref):
    h = jnp.dot(x_ref[...], wx_ref[...], preferred_element_type=jnp.float32)
    h = _swish(h + bx_ref[...])
    r = lax.dot_general(rbft_ref[...], wr_ref[...],
                        dimension_numbers=(((0,), (0,)), ((), ())),
                        preferred_element_type=jnp.float32)
    r = _swish(r + br_ref[...])
    out_ref[...] = h * r


def _edge_stage(x, rbf_t, W_x, b_x, W_rbf, b_rbf, off_blocks, n_blocks,
                block_e):
    E, EMB = x.shape
    RBF = rbf_t.shape[0]
    return pl.pallas_call(
        _edge_body,
        grid=(n_blocks,),
        in_specs=[
            pl.BlockSpec((block_e, EMB), lambda i: (off_blocks + i, 0)),
            pl.BlockSpec((RBF, block_e), lambda i: (0, off_blocks + i)),
            pl.BlockSpec((EMB, EMB), lambda i: (0, 0)),
            pl.BlockSpec((1, EMB), lambda i: (0, 0)),
            pl.BlockSpec((RBF, EMB), lambda i: (0, 0)),
            pl.BlockSpec((1, EMB), lambda i: (0, 0)),
        ],
        out_specs=pl.BlockSpec((block_e, EMB), lambda i: (i, 0)),
        out_shape=jax.ShapeDtypeStruct((n_blocks * block_e, EMB), jnp.float32),
    )(x, rbf_t, W_x, b_x.reshape(1, EMB), W_rbf, b_rbf.reshape(1, EMB))


def _seg_sum_sc(conv3, idnb, chunk_off):
    n_chunks, _, EMB = conv3.shape
    info = plsc.get_sparse_core_info()
    NC, NS = info.num_cores, info.num_subcores
    nw = NC * NS
    cpw = n_chunks // nw
    rows_per_s = _N_PAD // NS
    mesh = plsc.VectorSubcoreMesh(core_axis_name="c", subcore_axis_name="s")
    nbuf = 4
    n_groups = cpw // nbuf
    n_tail = cpw - n_groups * nbuf

    @functools.partial(
        pl.kernel,
        mesh=mesh,
        out_type=jax.ShapeDtypeStruct((NC, _N_PAD, EMB), jnp.float32),
        scratch_types=[
            pltpu.VMEM((nbuf, _CHUNK), jnp.int32),
            pltpu.VMEM((nbuf, _CHUNK, EMB), jnp.float32),
            pltpu.VMEM_SHARED((_N_PAD, EMB), jnp.float32),
            pltpu.SemaphoreType.DMA((nbuf,)),
            pltpu.SemaphoreType.DMA((nbuf,)),
            pltpu.SemaphoreType.DMA((nbuf,)),
        ],
    )
    def k(conv_hbm, idnb_hbm, out_hbm, idx_ring, bufs, acc, semf, semi,
          sems):
        c = lax.axis_index("c")
        s = lax.axis_index("s")
        w = c * NS + s
        base = w * cpw
        ebase = (chunk_off + w * cpw) * _CHUNK
        my_rows = pl.ds(s * rows_per_s, rows_per_s)
        zero16 = jnp.zeros((16,), jnp.float32)

        def zfill(r, carry):
            for c16 in range(EMB // 16):
                bufs[0, r, pl.ds(c16 * 16, 16)] = zero16
            return carry

        lax.fori_loop(0, _CHUNK, zfill, None)
        for r in range(rows_per_s // _CHUNK):
            pltpu.sync_copy(
                bufs.at[0], acc.at[pl.ds(s * rows_per_s + r * _CHUNK, _CHUNK)])
        plsc.subcore_barrier()

        for b in range(nbuf):
            pltpu.async_copy(conv_hbm.at[base + b], bufs.at[b], semf.at[b])
            pltpu.async_copy(idnb_hbm.at[pl.ds(ebase + b * _CHUNK, _CHUNK)],
                             idx_ring.at[b], semi.at[b])

        def body(g, carry):
            j0 = nbuf * g
            for b in range(nbuf):
                pltpu.make_async_copy(
                    conv_hbm.at[base], bufs.at[b], semf.at[b]).wait()
                pltpu.make_async_copy(
                    idnb_hbm.at[pl.ds(0, _CHUNK)], idx_ring.at[b],
                    semi.at[b]).wait()
                pltpu.async_copy(
                    bufs.at[b], acc.at[idx_ring.at[b]], sems.at[b],
                    add=True)
            for b in range(nbuf):
                pltpu.make_async_copy(
                    bufs.at[b], acc.at[idx_ring.at[b]], sems.at[b]).wait()
                nxt = j0 + nbuf + b

                @pl.when(nxt < cpw)
                def _():
                    pltpu.async_copy(
                        conv_hbm.at[base + nxt], bufs.at[b], semf.at[b])
                    pltpu.async_copy(
                        idnb_hbm.at[pl.ds(ebase + nxt * _CHUNK, _CHUNK)],
                        idx_ring.at[b], semi.at[b])
            return carry

        lax.fori_loop(0, n_groups, body, None)
        for t in range(n_tail):
            pltpu.make_async_copy(
                conv_hbm.at[base], bufs.at[t], semf.at[t]).wait()
            pltpu.make_async_copy(
                idnb_hbm.at[pl.ds(0, _CHUNK)], idx_ring.at[t],
                semi.at[t]).wait()
            pltpu.sync_copy(bufs.at[t], acc.at[idx_ring.at[t]], add=True)
        plsc.subcore_barrier()
        pltpu.sync_copy(acc.at[my_rows], out_hbm.at[c].at[my_rows])

    return k(conv3, idnb)


def _node_body(*refs):
    nk = len(_SPLITS)
    p_refs = refs[:nk]
    w1_ref, b1_ref, w2_ref, b2_ref, w3_ref, b3_ref, wf_ref, out_ref = refs[nk:]
    h = p_refs[0][0] + p_refs[0][1]
    for p in p_refs[1:]:
        h = h + p[0] + p[1]
    h = _swish(jnp.dot(h, w1_ref[...], preferred_element_type=jnp.float32)
               + b1_ref[...])
    h = _swish(jnp.dot(h, w2_ref[...], preferred_element_type=jnp.float32)
               + b2_ref[...])
    h = _swish(jnp.dot(h, w3_ref[...], preferred_element_type=jnp.float32)
               + b3_ref[...])
    out_ref[...] = lax.dot_general(wf_ref[...], h,
                                   dimension_numbers=(((0,), (1,)), ((), ())),
                                   preferred_element_type=jnp.float32)


def _node_stage(partials, W1, b1, W2, b2, W3, b3, W_final, n_out, block_n):
    NC, _, EMB = partials[0].shape
    nk = len(_SPLITS)
    grid = (n_out // block_n,)
    return pl.pallas_call(
        _node_body,
        grid=grid,
        in_specs=[
            pl.BlockSpec((NC, block_n, EMB), lambda i: (0, i, 0))
            for _ in range(nk)
        ] + [
            pl.BlockSpec((EMB, EMB), lambda i: (0, 0)),
            pl.BlockSpec((1, EMB), lambda i: (0, 0)),
            pl.BlockSpec((EMB, EMB), lambda i: (0, 0)),
            pl.BlockSpec((1, EMB), lambda i: (0, 0)),
            pl.BlockSpec((EMB, EMB), lambda i: (0, 0)),
            pl.BlockSpec((1, EMB), lambda i: (0, 0)),
            pl.BlockSpec((EMB, 1), lambda i: (0, 0)),
        ],
        out_specs=pl.BlockSpec((1, block_n), lambda i: (0, i)),
        out_shape=jax.ShapeDtypeStruct((1, n_out), jnp.float32),
    )(*partials, W1, b1.reshape(1, EMB), W2, b2.reshape(1, EMB),
      W3, b3.reshape(1, EMB), W_final)


def kernel(x, rbf, idnb_i, n_atoms, coef_rbf_a, coef_rbf_b, coef_x,
           coef_final, W_x, b_x, W_rbf, b_rbf, W1, b1, W2, b2, W3, b3,
           W_final):
    E, EMB = x.shape
    W_rbf_s = W_rbf * coef_rbf_a[0]
    W1_s = W1 * (coef_x[0] * coef_rbf_b[0])
    W_final_s = W_final * coef_final[0]

    nw = 32
    block_e = nw * _CHUNK
    rbf_t = rbf.T

    partials = []
    cum = 0
    for cpw in _SPLITS:
        mult = 2 if cpw % 2 == 0 else 1
        conv = _edge_stage(x, rbf_t, W_x, b_x, W_rbf_s, b_rbf,
                           off_blocks=cum // mult, n_blocks=cpw // mult,
                           block_e=block_e * mult)
        conv3 = conv.reshape(cpw * nw, _CHUNK, EMB)
        partials.append(_seg_sum_sc(conv3, idnb_i, chunk_off=cum * nw))
        cum += cpw

    out_t = _node_stage(partials, W1_s, b1, W2, b2, W3, b3, W_final_s,
                        n_out=_N_PAD, block_n=2048)
    return out_t[:, :_N_ATOMS].T
